# baseline (speedup 1.0000x reference)
import functools

import jax
import jax.numpy as jnp
from jax import lax
from jax.experimental import pallas as pl
from jax.experimental.pallas import tpu as pltpu
from jax.experimental.pallas import tpu_sc as plsc

NUM_CORES = 2
NUM_SUBCORES = 16
NUM_LANES = 16
NW = NUM_CORES * NUM_SUBCORES

L = 20
D = 64
GROUP = 32
IDX_CHUNK = 128
N_CHUNKS = GROUP * L // IDX_CHUNK


def _fire(table_hbm, tok_v, rows_buf, sem, g):
    g_tok = g * (GROUP * L)
    for q in range(N_CHUNKS):
        idx_ref = tok_v.at[pl.ds(g_tok + q * IDX_CHUNK, IDX_CHUNK)]
        dst = rows_buf.at[pl.ds(q * IDX_CHUNK, IDX_CHUNK)]
        pltpu.async_copy(table_hbm.at[idx_ref], dst, sem)


def _body(tok_hbm, lens_hbm, table_hbm, out_hbm, tok_v, lens_v, rows_a,
          rows_b, out_v, sem_a, sem_b, n_bags_per_w, n_groups):
    wid = lax.axis_index("s") * NUM_CORES + lax.axis_index("c")
    bag_base = wid * n_bags_per_w
    tok_base = bag_base * L

    pltpu.sync_copy(tok_hbm.at[pl.ds(tok_base, n_bags_per_w * L)], tok_v)
    pltpu.sync_copy(lens_hbm.at[pl.ds(bag_base, n_bags_per_w)], lens_v)

    lane = lax.broadcasted_iota(jnp.int32, (NUM_LANES,), 0)
    lane_l = lane * L

    def wait(rows_buf, sem):
        pltpu.make_async_copy(
            table_hbm.at[pl.ds(0, GROUP * L)], rows_buf, sem).wait()

    def compute(g, rows_buf):
        g_tok = g * (GROUP * L)
        for half in range(GROUP // NUM_LANES):
            h_tok = g_tok + half * (NUM_LANES * L)
            h_bag = g * GROUP + half * NUM_LANES
            lens_i = plsc.load_gather(lens_v, [h_bag + lane])
            inv = 1.0 / jnp.maximum(lens_i.astype(jnp.float32), 1.0)
            scales = []
            for j in range(L):
                t = plsc.load_gather(tok_v, [h_tok + j + lane_l])
                scales.append(jnp.where(t != 0, inv, 0.0))
            row0 = half * (NUM_LANES * L)
            orow0 = half * NUM_LANES

            def col_body(d, _):
                col = jnp.full((NUM_LANES,), 0, jnp.int32) + d
                acc = scales[0] * plsc.load_gather(
                    rows_buf, [row0 + lane_l, col])
                for j in range(1, L):
                    v = plsc.load_gather(rows_buf, [row0 + lane_l + j, col])
                    acc = acc + scales[j] * v
                plsc.store_scatter(out_v, [orow0 + lane, col], acc)
                return 0

            lax.fori_loop(0, D, col_body, 0, unroll=2)

        pltpu.sync_copy(out_v, out_hbm.at[pl.ds(bag_base + g * GROUP, GROUP)])

    _fire(table_hbm, tok_v, rows_a, sem_a, 0)

    def pair_body(i, _):
        g = i * 2

        @pl.when(g + 1 < n_groups)
        def _():
            _fire(table_hbm, tok_v, rows_b, sem_b, g + 1)
        wait(rows_a, sem_a)
        compute(g, rows_a)

        @pl.when(g + 2 < n_groups)
        def _():
            _fire(table_hbm, tok_v, rows_a, sem_a, g + 2)

        @pl.when(g + 1 < n_groups)
        def _():
            wait(rows_b, sem_b)
            compute(g + 1, rows_b)
        return 0

    lax.fori_loop(0, (n_groups + 1) // 2, pair_body, 0)


def kernel(token_ids, lengths, table):
    B, A, Ltok = token_ids.shape
    assert Ltok == L and table.shape[1] == D
    n_bags = B * A
    assert n_bags % (NW * GROUP) == 0
    n_bags_per_w = n_bags // NW
    n_groups = n_bags_per_w // GROUP

    tok_flat = token_ids.reshape(-1).astype(jnp.int32)
    lens_flat = lengths.reshape(-1).astype(jnp.int32)

    mesh = plsc.VectorSubcoreMesh(core_axis_name="c", subcore_axis_name="s")
    body = functools.partial(_body, n_bags_per_w=n_bags_per_w,
                             n_groups=n_groups)
    out = pl.kernel(
        body,
        out_type=jax.ShapeDtypeStruct((n_bags, D), jnp.float32),
        mesh=mesh,
        compiler_params=pltpu.CompilerParams(needs_layout_passes=False,
                                             use_tc_tiling_on_sc=False),
        scratch_types=[
            pltpu.VMEM((n_bags_per_w * L,), jnp.int32),
            pltpu.VMEM((n_bags_per_w,), jnp.int32),
            pltpu.VMEM((GROUP * L, D), jnp.float32),
            pltpu.VMEM((GROUP * L, D), jnp.float32),
            pltpu.VMEM((GROUP, D), jnp.float32),
            pltpu.SemaphoreType.DMA,
            pltpu.SemaphoreType.DMA,
        ],
    )(tok_flat, lens_flat, table)
    return out.reshape(B, A, D)


# speedup vs baseline: 1.0010x; 1.0010x over previous
import functools

import jax
import jax.numpy as jnp
from jax import lax
from jax.experimental import pallas as pl
from jax.experimental.pallas import tpu as pltpu
from jax.experimental.pallas import tpu_sc as plsc

NUM_CORES = 2
NUM_SUBCORES = 16
NUM_LANES = 16
NW = NUM_CORES * NUM_SUBCORES

L = 20
D = 64
GROUP = 32
IDX_CHUNK = 32
N_CHUNKS = GROUP * L // IDX_CHUNK


def _fire(table_hbm, tok_v, rows_buf, sem, g):
    g_tok = g * (GROUP * L)
    for q in range(N_CHUNKS):
        idx_ref = tok_v.at[pl.ds(g_tok + q * IDX_CHUNK, IDX_CHUNK)]
        dst = rows_buf.at[pl.ds(q * IDX_CHUNK, IDX_CHUNK)]
        pltpu.async_copy(table_hbm.at[idx_ref], dst, sem)


def _body(tok_hbm, lens_hbm, table_hbm, out_hbm, tok_v, lens_v, rows_a,
          rows_b, out_v, sem_a, sem_b, n_bags_per_w, n_groups):
    wid = lax.axis_index("s") * NUM_CORES + lax.axis_index("c")
    bag_base = wid * n_bags_per_w
    tok_base = bag_base * L

    pltpu.sync_copy(tok_hbm.at[pl.ds(tok_base, n_bags_per_w * L)], tok_v)
    pltpu.sync_copy(lens_hbm.at[pl.ds(bag_base, n_bags_per_w)], lens_v)

    lane = lax.broadcasted_iota(jnp.int32, (NUM_LANES,), 0)
    lane_l = lane * L

    def wait(rows_buf, sem):
        pltpu.make_async_copy(
            table_hbm.at[pl.ds(0, GROUP * L)], rows_buf, sem).wait()

    def compute(g, rows_buf):
        g_tok = g * (GROUP * L)
        for half in range(GROUP // NUM_LANES):
            h_tok = g_tok + half * (NUM_LANES * L)
            h_bag = g * GROUP + half * NUM_LANES
            lens_i = plsc.load_gather(lens_v, [h_bag + lane])
            inv = 1.0 / jnp.maximum(lens_i.astype(jnp.float32), 1.0)
            scales = []
            for j in range(L):
                t = plsc.load_gather(tok_v, [h_tok + j + lane_l])
                scales.append(jnp.where(t != 0, inv, 0.0))
            row0 = half * (NUM_LANES * L)
            orow0 = half * NUM_LANES

            def col_body(d, _):
                col = jnp.full((NUM_LANES,), 0, jnp.int32) + d
                acc = scales[0] * plsc.load_gather(
                    rows_buf, [row0 + lane_l, col])
                for j in range(1, L):
                    v = plsc.load_gather(rows_buf, [row0 + lane_l + j, col])
                    acc = acc + scales[j] * v
                plsc.store_scatter(out_v, [orow0 + lane, col], acc)
                return 0

            lax.fori_loop(0, D, col_body, 0, unroll=2)

        pltpu.sync_copy(out_v, out_hbm.at[pl.ds(bag_base + g * GROUP, GROUP)])

    _fire(table_hbm, tok_v, rows_a, sem_a, 0)

    def pair_body(i, _):
        g = i * 2

        @pl.when(g + 1 < n_groups)
        def _():
            _fire(table_hbm, tok_v, rows_b, sem_b, g + 1)
        wait(rows_a, sem_a)
        compute(g, rows_a)

        @pl.when(g + 2 < n_groups)
        def _():
            _fire(table_hbm, tok_v, rows_a, sem_a, g + 2)

        @pl.when(g + 1 < n_groups)
        def _():
            wait(rows_b, sem_b)
            compute(g + 1, rows_b)
        return 0

    lax.fori_loop(0, (n_groups + 1) // 2, pair_body, 0)


def kernel(token_ids, lengths, table):
    B, A, Ltok = token_ids.shape
    assert Ltok == L and table.shape[1] == D
    n_bags = B * A
    assert n_bags % (NW * GROUP) == 0
    n_bags_per_w = n_bags // NW
    n_groups = n_bags_per_w // GROUP

    tok_flat = token_ids.reshape(-1).astype(jnp.int32)
    lens_flat = lengths.reshape(-1).astype(jnp.int32)

    mesh = plsc.VectorSubcoreMesh(core_axis_name="c", subcore_axis_name="s")
    body = functools.partial(_body, n_bags_per_w=n_bags_per_w,
                             n_groups=n_groups)
    out = pl.kernel(
        body,
        out_type=jax.ShapeDtypeStruct((n_bags, D), jnp.float32),
        mesh=mesh,
        compiler_params=pltpu.CompilerParams(needs_layout_passes=False,
                                             use_tc_tiling_on_sc=False),
        scratch_types=[
            pltpu.VMEM((n_bags_per_w * L,), jnp.int32),
            pltpu.VMEM((n_bags_per_w,), jnp.int32),
            pltpu.VMEM((GROUP * L, D), jnp.float32),
            pltpu.VMEM((GROUP * L, D), jnp.float32),
            pltpu.VMEM((GROUP, D), jnp.float32),
            pltpu.SemaphoreType.DMA,
            pltpu.SemaphoreType.DMA,
        ],
    )(tok_flat, lens_flat, table)
    return out.reshape(B, A, D)


# speedup vs baseline: 1.8397x; 1.8379x over previous
import functools

import jax
import jax.numpy as jnp
from jax import lax
from jax.experimental import pallas as pl
from jax.experimental.pallas import tpu as pltpu
from jax.experimental.pallas import tpu_sc as plsc

NUM_CORES = 2
NUM_SUBCORES = 16
NUM_LANES = 16
NW = NUM_CORES * NUM_SUBCORES

L = 20
D = 64
NCB = D // NUM_LANES
GROUP = 32
IDX_CHUNK = 32
N_CHUNKS = GROUP * L // IDX_CHUNK


def _fire(table_hbm, tok_v, rows_buf, sem, g):
    g_tok = g * (GROUP * L)
    for q in range(N_CHUNKS):
        idx_ref = tok_v.at[pl.ds(g_tok + q * IDX_CHUNK, IDX_CHUNK)]
        dst = rows_buf.at[pl.ds(q * IDX_CHUNK, IDX_CHUNK)]
        pltpu.async_copy(table_hbm.at[idx_ref], dst, sem)


def _body(tok_hbm, lens_hbm, table_hbm, out_hbm, tok_v, lens_v, rows_a,
          rows_b, out_v, sc_v, t0_v, sem_a, sem_b, n_bags_per_w, n_groups):
    wid = lax.axis_index("s") * NUM_CORES + lax.axis_index("c")
    bag_base = wid * n_bags_per_w
    tok_base = bag_base * L

    pltpu.sync_copy(tok_hbm.at[pl.ds(tok_base, n_bags_per_w * L)], tok_v)
    pltpu.sync_copy(lens_hbm.at[pl.ds(bag_base, n_bags_per_w)], lens_v)
    pltpu.sync_copy(table_hbm.at[pl.ds(0, 1)], t0_v)

    lane = lax.broadcasted_iota(jnp.int32, (NUM_LANES,), 0)
    lane_l = lane * L
    t0 = [t0_v[0, pl.ds(c * NUM_LANES, NUM_LANES)] for c in range(NCB)]

    def wait(rows_buf, sem):
        pltpu.make_async_copy(
            table_hbm.at[pl.ds(0, GROUP * L)], rows_buf, sem).wait()

    def compute(g, rows_buf):
        g_tok = g * (GROUP * L)
        for half in range(GROUP // NUM_LANES):
            lens_i = plsc.load_gather(
                lens_v, [g * GROUP + half * NUM_LANES + lane])
            inv = 1.0 / jnp.maximum(lens_i.astype(jnp.float32), 1.0)
            h_tok = g_tok + half * (NUM_LANES * L)
            z = jnp.zeros((NUM_LANES,), jnp.float32)
            for j in range(L):
                t = plsc.load_gather(tok_v, [h_tok + j + lane_l])
                z = z + jnp.where(t == 0, 1.0, 0.0)
            plsc.store_scatter(sc_v, [half * NUM_LANES + lane], z * inv)
            plsc.store_scatter(sc_v, [GROUP + half * NUM_LANES + lane], inv)

        def bag_body(r, _):
            base = r * L
            accs = [rows_buf[base, pl.ds(c * NUM_LANES, NUM_LANES)]
                    for c in range(NCB)]
            for j in range(1, L):
                for c in range(NCB):
                    v = rows_buf[base + j, pl.ds(c * NUM_LANES, NUM_LANES)]
                    accs[c] = accs[c] + v
            rvec = jnp.full((NUM_LANES,), 0, jnp.int32) + r
            cv = plsc.load_gather(sc_v, [rvec])
            iv = plsc.load_gather(sc_v, [rvec + GROUP])
            for c in range(NCB):
                out_v[r, pl.ds(c * NUM_LANES, NUM_LANES)] = (
                    (accs[c] - cv * t0[c]) * iv)
            return 0

        lax.fori_loop(0, GROUP, bag_body, 0)
        pltpu.sync_copy(out_v, out_hbm.at[pl.ds(bag_base + g * GROUP, GROUP)])

    _fire(table_hbm, tok_v, rows_a, sem_a, 0)

    def pair_body(i, _):
        g = i * 2

        @pl.when(g + 1 < n_groups)
        def _():
            _fire(table_hbm, tok_v, rows_b, sem_b, g + 1)
        wait(rows_a, sem_a)
        compute(g, rows_a)

        @pl.when(g + 2 < n_groups)
        def _():
            _fire(table_hbm, tok_v, rows_a, sem_a, g + 2)

        @pl.when(g + 1 < n_groups)
        def _():
            wait(rows_b, sem_b)
            compute(g + 1, rows_b)
        return 0

    lax.fori_loop(0, (n_groups + 1) // 2, pair_body, 0)


def kernel(token_ids, lengths, table):
    B, A, Ltok = token_ids.shape
    assert Ltok == L and table.shape[1] == D
    n_bags = B * A
    assert n_bags % (NW * GROUP) == 0
    n_bags_per_w = n_bags // NW
    n_groups = n_bags_per_w // GROUP

    tok_flat = token_ids.reshape(-1).astype(jnp.int32)
    lens_flat = lengths.reshape(-1).astype(jnp.int32)

    mesh = plsc.VectorSubcoreMesh(core_axis_name="c", subcore_axis_name="s")
    body = functools.partial(_body, n_bags_per_w=n_bags_per_w,
                             n_groups=n_groups)
    out = pl.kernel(
        body,
        out_type=jax.ShapeDtypeStruct((n_bags, D), jnp.float32),
        mesh=mesh,
        compiler_params=pltpu.CompilerParams(needs_layout_passes=False,
                                             use_tc_tiling_on_sc=False),
        scratch_types=[
            pltpu.VMEM((n_bags_per_w * L,), jnp.int32),
            pltpu.VMEM((n_bags_per_w,), jnp.int32),
            pltpu.VMEM((GROUP * L, D), jnp.float32),
            pltpu.VMEM((GROUP * L, D), jnp.float32),
            pltpu.VMEM((GROUP, D), jnp.float32),
            pltpu.VMEM((2 * GROUP,), jnp.float32),
            pltpu.VMEM((1, D), jnp.float32),
            pltpu.SemaphoreType.DMA,
            pltpu.SemaphoreType.DMA,
        ],
    )(tok_flat, lens_flat, table)
    return out.reshape(B, A, D)
